# baseline (speedup 1.0000x reference)
import functools

import jax
import jax.numpy as jnp
from jax import lax
from jax.experimental import pallas as pl
from jax.experimental.pallas import tpu as pltpu
from jax.experimental.pallas import tpu_sc as plsc

N_NODES = 10000
N_EDGES = 320000
D = 128

NC = 2
NS = 16
NW = NC * NS

K = 128
CPW = 80
NSLAB = 2
CPS = CPW // NSLAB
E_PAD = NW * CPW * K
RPT = 632
N_ACC = NS * RPT

_sc_mesh = plsc.VectorSubcoreMesh(core_axis_name="c", subcore_axis_name="s")


@functools.partial(
    pl.kernel,
    out_type=jax.ShapeDtypeStruct((NC, N_ACC, D), jnp.float32),
    mesh=_sc_mesh,
    scratch_types=[
        pltpu.VMEM_SHARED((N_ACC, D), jnp.float32),
        pltpu.VMEM((CPS, K), jnp.int32),
        pltpu.VMEM((CPS, K), jnp.int32),
        pltpu.VMEM((2, K, D), jnp.float32),
        pltpu.SemaphoreType.DMA,
        pltpu.SemaphoreType.DMA,
    ],
)
def _sc_scatter(x_hbm, src_hbm, dst_hbm, zeros_hbm, out_hbm,
                acc, src_v, dst_v, rows_v, sem0, sem1):
    c = lax.axis_index("c")
    s = lax.axis_index("s")
    wid = c * NS + s

    pltpu.sync_copy(zeros_hbm, acc.at[pl.ds(s * RPT, RPT)])
    plsc.subcore_barrier()

    sems = (sem0, sem1)
    for p in range(NSLAB):
        pltpu.sync_copy(src_hbm.at[wid, p], src_v)
        pltpu.sync_copy(dst_hbm.at[wid, p], dst_v)
        pltpu.async_copy(x_hbm.at[src_v.at[0]], rows_v.at[0], sem0)
        pltpu.async_copy(x_hbm.at[src_v.at[1]], rows_v.at[1], sem1)

        def pair(i, carry):
            jj = 2 * i
            for bidx in range(2):
                j = jj + bidx
                buf = rows_v.at[bidx]
                pltpu.make_async_copy(
                    x_hbm.at[src_v.at[j]], buf, sems[bidx]).wait()
                pltpu.sync_copy(buf, acc.at[dst_v.at[j]], add=True)

                @pl.when(j + 2 < CPS)
                def _():
                    pltpu.async_copy(x_hbm.at[src_v.at[j + 2]], buf, sems[bidx])
            return carry

        lax.fori_loop(0, CPS // 2, pair, 0)
    plsc.subcore_barrier()

    pltpu.sync_copy(acc.at[pl.ds(s * RPT, RPT)],
                    out_hbm.at[c].at[pl.ds(s * RPT, RPT)])


_TC_BLK = 1000


def _linear_body(pa_ref, pb_ref, w_ref, b_ref, o_ref):
    agg = pa_ref[0] + pb_ref[0]
    o_ref[...] = lax.dot_general(
        agg, w_ref[...], (((1,), (1,)), ((), ())),
        preferred_element_type=jnp.float32) + b_ref[...]


def _tc_linear(partials, w, b):
    b2 = b.reshape(1, D)
    return pl.pallas_call(
        _linear_body,
        grid=(N_NODES // _TC_BLK,),
        in_specs=[
            pl.BlockSpec((1, _TC_BLK, D), lambda i: (0, i, 0)),
            pl.BlockSpec((1, _TC_BLK, D), lambda i: (1, i, 0)),
            pl.BlockSpec((D, D), lambda i: (0, 0)),
            pl.BlockSpec((1, D), lambda i: (0, 0)),
        ],
        out_specs=pl.BlockSpec((_TC_BLK, D), lambda i: (i, 0)),
        out_shape=jax.ShapeDtypeStruct((N_NODES, D), jnp.float32),
    )(partials, partials, w, b2)


def kernel(x, edge_index, W, b):
    src = edge_index[0]
    dst = edge_index[1]
    pad = E_PAD - N_EDGES
    src_p = jnp.pad(src, (0, pad)).reshape(NW, NSLAB, CPS, K)
    dst_p = jnp.pad(dst, (0, pad),
                    constant_values=N_NODES).reshape(NW, NSLAB, CPS, K)
    zeros = jnp.zeros((RPT, D), jnp.float32)
    partials = _sc_scatter(x, src_p, dst_p, zeros)
    return _tc_linear(partials, W, b)


# speedup vs baseline: 1.1267x; 1.1267x over previous
import functools

import jax
import jax.numpy as jnp
from jax import lax
from jax.experimental import pallas as pl
from jax.experimental.pallas import tpu as pltpu
from jax.experimental.pallas import tpu_sc as plsc

N_NODES = 10000
N_EDGES = 320000
D = 128

NC = 2
NS = 16
HN = N_NODES // NC
HROWS = 5120
HRPT = HROWS // NS

K = 64
CPS = 4
SLABS = 80
E_PAD = NS * SLABS * CPS * K
RPT = 632
N_ACC = NS * RPT
DUMMY = N_NODES

_sc_mesh = plsc.VectorSubcoreMesh(core_axis_name="c", subcore_axis_name="s")


@functools.partial(
    pl.kernel,
    out_type=jax.ShapeDtypeStruct((NC, N_ACC, D), jnp.float32),
    mesh=_sc_mesh,
    scratch_types=[
        pltpu.VMEM_SHARED((HROWS, D), jnp.float32),
        pltpu.VMEM_SHARED((N_ACC, D), jnp.float32),
        pltpu.VMEM((2, CPS, K), jnp.int32),
        pltpu.VMEM((K, D), jnp.float32),
        pltpu.SemaphoreType.DMA,
    ],
)
def _sc_scatter(xh_hbm, idx_hbm, zeros_hbm, out_hbm,
                x_sp, acc, idx_v, rows_v, sem):
    c = lax.axis_index("c")
    s = lax.axis_index("s")
    lo = c * HN

    pltpu.sync_copy(zeros_hbm, acc.at[pl.ds(s * RPT, RPT)])
    pltpu.sync_copy(xh_hbm.at[c].at[pl.ds(s * HRPT, HRPT)],
                    x_sp.at[pl.ds(s * HRPT, HRPT)])
    plsc.subcore_barrier()

    def slab(sl, carry):
        pltpu.sync_copy(idx_hbm.at[s, sl], idx_v)
        for r in range(CPS):
            for g in range(K // 16):
                sv = idx_v[0, r, pl.ds(g * 16, 16)]
                dv = idx_v[1, r, pl.ds(g * 16, 16)]
                m = (sv >= lo) & (sv < lo + HN)
                idx_v[0, r, pl.ds(g * 16, 16)] = jnp.where(m, sv - lo, 0)
                idx_v[1, r, pl.ds(g * 16, 16)] = jnp.where(m, dv, DUMMY)
        for j in range(CPS):
            pltpu.async_copy(x_sp.at[idx_v.at[0, j]], rows_v, sem).wait()
            pltpu.sync_copy(rows_v, acc.at[idx_v.at[1, j]], add=True)
        return carry

    lax.fori_loop(0, SLABS, slab, 0)
    plsc.subcore_barrier()

    pltpu.sync_copy(acc.at[pl.ds(s * RPT, RPT)],
                    out_hbm.at[c].at[pl.ds(s * RPT, RPT)])


_TC_BLK = 1000


def _linear_body(pa_ref, pb_ref, w_ref, b_ref, o_ref):
    agg = pa_ref[0] + pb_ref[0]
    o_ref[...] = lax.dot_general(
        agg, w_ref[...], (((1,), (1,)), ((), ())),
        preferred_element_type=jnp.float32) + b_ref[...]


def _tc_linear(partials, w, b):
    b2 = b.reshape(1, D)
    return pl.pallas_call(
        _linear_body,
        grid=(N_NODES // _TC_BLK,),
        in_specs=[
            pl.BlockSpec((1, _TC_BLK, D), lambda i: (0, i, 0)),
            pl.BlockSpec((1, _TC_BLK, D), lambda i: (1, i, 0)),
            pl.BlockSpec((D, D), lambda i: (0, 0)),
            pl.BlockSpec((1, D), lambda i: (0, 0)),
        ],
        out_specs=pl.BlockSpec((_TC_BLK, D), lambda i: (i, 0)),
        out_shape=jax.ShapeDtypeStruct((N_NODES, D), jnp.float32),
    )(partials, partials, w, b2)


def kernel(x, edge_index, W, b):
    src = edge_index[0]
    dst = edge_index[1]
    pad = E_PAD - N_EDGES
    src_p = jnp.pad(src, (0, pad)).reshape(NS, SLABS, CPS, K)
    dst_p = jnp.pad(dst, (0, pad), constant_values=DUMMY).reshape(
        NS, SLABS, CPS, K)
    idx = jnp.stack([src_p, dst_p], axis=2)
    xh = jnp.pad(x.reshape(NC, HN, D), ((0, 0), (0, HROWS - HN), (0, 0)))
    zeros = jnp.zeros((RPT, D), jnp.float32)
    partials = _sc_scatter(xh, idx, zeros)
    return _tc_linear(partials, W, b)
